# Initial kernel scaffold; baseline (speedup 1.0000x reference)
import jax
import jax.numpy as jnp
from jax.experimental import pallas as pl


def kernel(x, W_t):
    raise NotImplementedError("write your pallas kernel here")


# speedup vs baseline: 1.2223x; 1.2223x over previous
import functools

import jax
import jax.numpy as jnp
from jax.experimental import pallas as pl

D_MODEL = 2048
N_EXPERTS = 64
TOP_K = 8
BLOCK_TOKENS = 1024


def _router_block(x_ref, w_ref, out_w_ref, out_i_ref):
    x = x_ref[...]
    w = w_ref[...]
    logits = jnp.dot(x, w, preferred_element_type=jnp.float32)

    m = jnp.max(logits, axis=-1, keepdims=True)
    e = jnp.exp(logits - m)
    probs = e / jnp.sum(e, axis=-1, keepdims=True)

    lane = jax.lax.broadcasted_iota(jnp.int32, probs.shape, 1)
    vals = []
    idxs = []
    p = probs
    for _ in range(TOP_K):
        mk = jnp.max(p, axis=-1, keepdims=True)
        ik = jnp.min(jnp.where(p == mk, lane, N_EXPERTS), axis=-1,
                     keepdims=True)
        vals.append(mk)
        idxs.append(ik)
        p = jnp.where(lane == ik, -1.0, p)

    top_w = jnp.concatenate(vals, axis=-1)
    top_i = jnp.concatenate(idxs, axis=-1)
    top_w = top_w / (jnp.sum(top_w, axis=-1, keepdims=True) + 1e-9)

    out_w_ref[...] = top_w
    out_i_ref[...] = top_i


@functools.partial(jax.jit, static_argnames=())
def kernel(x, W_t):
    n_tokens = x.shape[0]
    grid = (n_tokens // BLOCK_TOKENS,)
    out_w, out_i = pl.pallas_call(
        _router_block,
        grid=grid,
        in_specs=[
            pl.BlockSpec((BLOCK_TOKENS, D_MODEL), lambda i: (i, 0)),
            pl.BlockSpec((D_MODEL, N_EXPERTS), lambda i: (0, 0)),
        ],
        out_specs=[
            pl.BlockSpec((BLOCK_TOKENS, TOP_K), lambda i: (i, 0)),
            pl.BlockSpec((BLOCK_TOKENS, TOP_K), lambda i: (i, 0)),
        ],
        out_shape=[
            jax.ShapeDtypeStruct((n_tokens, TOP_K), jnp.float32),
            jax.ShapeDtypeStruct((n_tokens, TOP_K), jnp.int32),
        ],
    )(x, W_t)
    return out_w, out_i.astype(jnp.int64)


# speedup vs baseline: 1.7848x; 1.4602x over previous
import functools

import jax
import jax.numpy as jnp
from jax.experimental import pallas as pl

D_MODEL = 2048
N_EXPERTS = 64
TOP_K = 8
BLOCK_TOKENS = 1024


def _router_block(x_ref, w_ref, out_w_ref, out_i_ref):
    x = x_ref[...]
    w = w_ref[...]
    logits = jax.lax.dot_general(
        w, x, (((0,), (1,)), ((), ())),
        preferred_element_type=jnp.float32)

    m = jnp.max(logits, axis=0, keepdims=True)
    e = jnp.exp(logits - m)
    s = jnp.sum(e, axis=0, keepdims=True)
    probs = e / s

    lane = jax.lax.broadcasted_iota(jnp.int32, e.shape, 0).astype(jnp.float32)
    vals = []
    idxs = []
    p = probs
    for _ in range(TOP_K):
        mk = jnp.max(p, axis=0, keepdims=True)
        ik = jnp.min(jnp.where(p == mk, lane, float(N_EXPERTS)), axis=0,
                     keepdims=True)
        vals.append(mk)
        idxs.append(ik)
        p = jnp.where(lane == ik, -1.0, p)

    top_w = jnp.concatenate(vals, axis=0)
    top_i = jnp.concatenate(idxs, axis=0)
    top_w = top_w / (jnp.sum(top_w, axis=0, keepdims=True) + 1e-9)

    out_w_ref[...] = top_w.T
    out_i_ref[...] = top_i.T.astype(jnp.int32)


@functools.partial(jax.jit, static_argnames=())
def kernel(x, W_t):
    n_tokens = x.shape[0]
    grid = (n_tokens // BLOCK_TOKENS,)
    out_w, out_i = pl.pallas_call(
        _router_block,
        grid=grid,
        in_specs=[
            pl.BlockSpec((BLOCK_TOKENS, D_MODEL), lambda i: (i, 0)),
            pl.BlockSpec((D_MODEL, N_EXPERTS), lambda i: (0, 0)),
        ],
        out_specs=[
            pl.BlockSpec((BLOCK_TOKENS, TOP_K), lambda i: (i, 0)),
            pl.BlockSpec((BLOCK_TOKENS, TOP_K), lambda i: (i, 0)),
        ],
        out_shape=[
            jax.ShapeDtypeStruct((n_tokens, TOP_K), jnp.float32),
            jax.ShapeDtypeStruct((n_tokens, TOP_K), jnp.int32),
        ],
    )(x, W_t)
    return out_w, out_i.astype(jnp.int64)


# speedup vs baseline: 1.8745x; 1.0503x over previous
import functools

import jax
import jax.numpy as jnp
from jax.experimental import pallas as pl

D_MODEL = 2048
N_EXPERTS = 64
TOP_K = 8
BLOCK_TOKENS = 2048


def _router_block(x_ref, w_ref, out_w_ref, out_i_ref):
    x = x_ref[...]
    w = w_ref[...]
    logits = jax.lax.dot_general(
        w, x, (((0,), (1,)), ((), ())),
        preferred_element_type=jnp.float32)

    m = jnp.max(logits, axis=0, keepdims=True)
    e = jnp.exp(logits - m)
    s = jnp.sum(e, axis=0, keepdims=True)
    probs = e / s

    lane = jax.lax.broadcasted_iota(jnp.int32, e.shape, 0).astype(jnp.float32)
    vals = []
    idxs = []
    p = probs
    for _ in range(TOP_K):
        mk = jnp.max(p, axis=0, keepdims=True)
        ik = jnp.min(jnp.where(p == mk, lane, float(N_EXPERTS)), axis=0,
                     keepdims=True)
        vals.append(mk)
        idxs.append(ik)
        p = jnp.where(lane == ik, -1.0, p)

    top_w = jnp.concatenate(vals, axis=0)
    top_i = jnp.concatenate(idxs, axis=0)
    top_w = top_w / (jnp.sum(top_w, axis=0, keepdims=True) + 1e-9)

    out_w_ref[...] = top_w.T
    out_i_ref[...] = top_i.T.astype(jnp.int32)


@functools.partial(jax.jit, static_argnames=())
def kernel(x, W_t):
    n_tokens = x.shape[0]
    grid = (n_tokens // BLOCK_TOKENS,)
    out_w, out_i = pl.pallas_call(
        _router_block,
        grid=grid,
        in_specs=[
            pl.BlockSpec((BLOCK_TOKENS, D_MODEL), lambda i: (i, 0)),
            pl.BlockSpec((D_MODEL, N_EXPERTS), lambda i: (0, 0)),
        ],
        out_specs=[
            pl.BlockSpec((BLOCK_TOKENS, TOP_K), lambda i: (i, 0)),
            pl.BlockSpec((BLOCK_TOKENS, TOP_K), lambda i: (i, 0)),
        ],
        out_shape=[
            jax.ShapeDtypeStruct((n_tokens, TOP_K), jnp.float32),
            jax.ShapeDtypeStruct((n_tokens, TOP_K), jnp.int32),
        ],
    )(x, W_t)
    return out_w, out_i.astype(jnp.int64)


# speedup vs baseline: 2.4405x; 1.3019x over previous
import jax
import jax.numpy as jnp
from jax.experimental import pallas as pl
from jax.experimental.pallas import tpu as pltpu

D_MODEL = 2048
N_EXPERTS = 64
TOP_K = 8
BLOCK_TOKENS = 2048


def _router_body(x_hbm, w_ref, out_w_ref, out_i_ref, xbuf, sem):
    n_tokens = x_hbm.shape[0]
    n_blocks = n_tokens // BLOCK_TOKENS
    w = w_ref[...]

    def x_copy(i, slot):
        return pltpu.make_async_copy(
            x_hbm.at[pl.ds(i * BLOCK_TOKENS, BLOCK_TOKENS), :],
            xbuf.at[slot],
            sem.at[slot],
        )

    x_copy(0, 0).start()
    for i in range(n_blocks):
        if i + 1 < n_blocks:
            x_copy(i + 1, (i + 1) % 2).start()
        x_copy(i, i % 2).wait()
        x = xbuf[i % 2]

        logits = jax.lax.dot_general(
            w, x, (((0,), (1,)), ((), ())),
            preferred_element_type=jnp.float32)

        m = jnp.max(logits, axis=0, keepdims=True)
        e = jnp.exp(logits - m)
        s = jnp.sum(e, axis=0, keepdims=True)
        probs = e / s

        lane = jax.lax.broadcasted_iota(
            jnp.int32, probs.shape, 0).astype(jnp.float32)
        vals = []
        idxs = []
        p = probs
        for k in range(TOP_K):
            mk = jnp.max(p, axis=0, keepdims=True)
            ik = jnp.min(jnp.where(p == mk, lane, float(N_EXPERTS)),
                         axis=0, keepdims=True)
            vals.append(mk)
            idxs.append(ik)
            if k + 1 < TOP_K:
                p = jnp.where(lane == ik, -1.0, p)

        top_w = jnp.concatenate(vals, axis=0)
        top_i = jnp.concatenate(idxs, axis=0)
        top_w = top_w / (jnp.sum(top_w, axis=0, keepdims=True) + 1e-9)

        cols = pl.ds(i * BLOCK_TOKENS, BLOCK_TOKENS)
        out_w_ref[:, cols] = top_w
        out_i_ref[:, cols] = top_i.astype(jnp.int32)


def kernel(x, W_t):
    n_tokens = x.shape[0]
    out_w_t, out_i_t = pl.pallas_call(
        _router_body,
        in_specs=[
            pl.BlockSpec(memory_space=pltpu.HBM),
            pl.BlockSpec(memory_space=pltpu.VMEM),
        ],
        out_specs=[
            pl.BlockSpec(memory_space=pltpu.VMEM),
            pl.BlockSpec(memory_space=pltpu.VMEM),
        ],
        out_shape=[
            jax.ShapeDtypeStruct((TOP_K, n_tokens), jnp.float32),
            jax.ShapeDtypeStruct((TOP_K, n_tokens), jnp.int32),
        ],
        scratch_shapes=[
            pltpu.VMEM((2, BLOCK_TOKENS, D_MODEL), jnp.float32),
            pltpu.SemaphoreType.DMA((2,)),
        ],
    )(x, W_t)
    return out_w_t.T, out_i_t.T.astype(jnp.int64)


# speedup vs baseline: 2.5006x; 1.0246x over previous
import jax
import jax.numpy as jnp
from jax.experimental import pallas as pl
from jax.experimental.pallas import tpu as pltpu

D_MODEL = 2048
N_EXPERTS = 64
TOP_K = 8
BLOCK_TOKENS = 1024


def _router_body(x_hbm, w_ref, out_w_ref, out_i_ref, xbuf, sem):
    n_tokens = x_hbm.shape[0]
    n_blocks = n_tokens // BLOCK_TOKENS
    w = w_ref[...]

    def x_copy(i, slot):
        return pltpu.make_async_copy(
            x_hbm.at[pl.ds(i * BLOCK_TOKENS, BLOCK_TOKENS), :],
            xbuf.at[slot],
            sem.at[slot],
        )

    x_copy(0, 0).start()
    for i in range(n_blocks):
        if i + 1 < n_blocks:
            x_copy(i + 1, (i + 1) % 2).start()
        x_copy(i, i % 2).wait()
        x = xbuf[i % 2]

        logits = jax.lax.dot_general(
            w, x, (((0,), (1,)), ((), ())),
            preferred_element_type=jnp.float32)

        m = jnp.max(logits, axis=0, keepdims=True)
        e = jnp.exp(logits - m)
        s = jnp.sum(e, axis=0, keepdims=True)
        probs = e / s

        lane = jax.lax.broadcasted_iota(
            jnp.int32, probs.shape, 0).astype(jnp.float32)
        vals = []
        idxs = []
        p = probs
        for k in range(TOP_K):
            mk = jnp.max(p, axis=0, keepdims=True)
            ik = jnp.min(jnp.where(p == mk, lane, float(N_EXPERTS)),
                         axis=0, keepdims=True)
            vals.append(mk)
            idxs.append(ik)
            if k + 1 < TOP_K:
                p = jnp.where(lane == ik, -1.0, p)

        top_w = jnp.concatenate(vals, axis=0)
        top_i = jnp.concatenate(idxs, axis=0)
        top_w = top_w / (jnp.sum(top_w, axis=0, keepdims=True) + 1e-9)

        cols = pl.ds(i * BLOCK_TOKENS, BLOCK_TOKENS)
        out_w_ref[:, cols] = top_w
        out_i_ref[:, cols] = top_i.astype(jnp.int32)


def kernel(x, W_t):
    n_tokens = x.shape[0]
    out_w_t, out_i_t = pl.pallas_call(
        _router_body,
        in_specs=[
            pl.BlockSpec(memory_space=pltpu.HBM),
            pl.BlockSpec(memory_space=pltpu.VMEM),
        ],
        out_specs=[
            pl.BlockSpec(memory_space=pltpu.VMEM),
            pl.BlockSpec(memory_space=pltpu.VMEM),
        ],
        out_shape=[
            jax.ShapeDtypeStruct((TOP_K, n_tokens), jnp.float32),
            jax.ShapeDtypeStruct((TOP_K, n_tokens), jnp.int32),
        ],
        scratch_shapes=[
            pltpu.VMEM((2, BLOCK_TOKENS, D_MODEL), jnp.float32),
            pltpu.SemaphoreType.DMA((2,)),
        ],
    )(x, W_t)
    return out_w_t.T, out_i_t.T.astype(jnp.int64)
